# Initial kernel scaffold; baseline (speedup 1.0000x reference)
import jax
import jax.numpy as jnp
from jax.experimental import pallas as pl


def kernel(station_index, event_index, phase_type, event_loc_w, event_time_w, station_dt_w, station_loc):
    raise NotImplementedError("write your pallas kernel here")


# speedup vs baseline: 58.4297x; 58.4297x over previous
import functools

import jax
import jax.numpy as jnp
from jax import lax
from jax.experimental import pallas as pl
from jax.experimental.pallas import tpu as pltpu
from jax.experimental.pallas import tpu_sc as plsc

_VP = 6.0
_VS = 6.0 / 1.73

_BLK = 2000
_CH = 100
_NCH = _BLK // _CH
_LANES = 16


def _tt_body(sta_i_hbm, ev_i_hbm, ph_hbm, table_hbm, sta_tab_hbm, out_hbm,
             evi_v, stai_v, ph_v, rows_v, sta_v, out_v, sem,
             *, num_blocks, num_workers):
    wid = lax.axis_index("s") * 2 + lax.axis_index("c")

    pltpu.sync_copy(sta_tab_hbm, sta_v)

    max_blocks_per_worker = (num_blocks + num_workers - 1) // num_workers

    def block_step(k, carry):
        blk = wid + num_workers * k

        @pl.when(blk < num_blocks)
        def _():
            base = blk * _BLK
            pltpu.sync_copy(ev_i_hbm.at[pl.ds(blk * _NCH, _NCH)], evi_v)
            pltpu.sync_copy(sta_i_hbm.at[pl.ds(base, _BLK)], stai_v)
            pltpu.sync_copy(ph_hbm.at[pl.ds(base, _BLK)], ph_v)
            copies = [
                pltpu.async_copy(table_hbm.at[evi_v.at[j]],
                                 rows_v.at[pl.ds(j * _CH, _CH)], sem)
                for j in range(_NCH)
            ]
            for c in copies:
                c.wait()

            def group_step(g, carry2):
                lane = lax.iota(jnp.int32, _LANES)
                rows16 = g * _LANES + lane
                c0 = jnp.zeros((_LANES,), jnp.int32)
                sta16 = stai_v[pl.ds(g * _LANES, _LANES)]
                ph16 = ph_v[pl.ds(g * _LANES, _LANES)]
                sbase = sta16 * 4
                ex = plsc.load_gather(rows_v, [rows16, c0])
                ey = plsc.load_gather(rows_v, [rows16, c0 + 1])
                ez = plsc.load_gather(rows_v, [rows16, c0 + 2])
                et = plsc.load_gather(rows_v, [rows16, c0 + 3])
                sx = plsc.load_gather(sta_v, [sbase])
                sy = plsc.load_gather(sta_v, [sbase + 1])
                sz = plsc.load_gather(sta_v, [sbase + 2])
                sdt = plsc.load_gather(sta_v, [sbase + 3])
                dx = ex - sx
                dy = ey - sy
                dz = ez - sz
                d2 = dx * dx + dy * dy + dz * dz
                seed = lax.bitcast_convert_type(
                    0x5F3759DF - lax.shift_right_logical(
                        lax.bitcast_convert_type(d2, jnp.int32), 1),
                    jnp.float32)
                y = seed * (1.5 - 0.5 * d2 * seed * seed)
                y = y * (1.5 - 0.5 * d2 * y * y)
                y = y * (1.5 - 0.5 * d2 * y * y)
                dist = jnp.where(d2 > 0.0, d2 * y, 0.0)
                inv_v = (1.0 / _VP) + ph16.astype(jnp.float32) * (1.0 / _VS - 1.0 / _VP)
                out_v[pl.ds(g * _LANES, _LANES)] = et + dist * inv_v + sdt
                return carry2

            lax.fori_loop(0, _BLK // _LANES, group_step, 0)
            pltpu.sync_copy(out_v, out_hbm.at[pl.ds(base, _BLK)])

        return carry

    lax.fori_loop(0, max_blocks_per_worker, block_step, 0)


@jax.jit
def _tt_pallas(station_index, event_index, phase_type, table, sta_tab):
    n = station_index.shape[0]
    assert n % _BLK == 0
    num_blocks = n // _BLK
    num_workers = 32
    mesh = plsc.VectorSubcoreMesh(core_axis_name="c", subcore_axis_name="s")
    body = functools.partial(_tt_body, num_blocks=num_blocks,
                             num_workers=num_workers)
    return pl.kernel(
        body,
        mesh=mesh,
        compiler_params=pltpu.CompilerParams(needs_layout_passes=False,
                                             use_tc_tiling_on_sc=False),
        out_type=jax.ShapeDtypeStruct((n,), jnp.float32),
        scratch_types=[
            pltpu.VMEM((_NCH, _CH), jnp.int32),
            pltpu.VMEM((_BLK,), jnp.int32),
            pltpu.VMEM((_BLK,), jnp.int32),
            pltpu.VMEM((_BLK, 16), jnp.float32),
            pltpu.VMEM((400,), jnp.float32),
            pltpu.VMEM((_BLK,), jnp.float32),
            pltpu.SemaphoreType.DMA,
        ],
    )(station_index, event_index, phase_type, table, sta_tab)


def kernel(station_index, event_index, phase_type, event_loc_w, event_time_w,
           station_dt_w, station_loc):
    table = jnp.pad(jnp.concatenate([event_loc_w, event_time_w], axis=1),
                    ((0, 0), (0, 12)))
    sta_tab = jnp.concatenate([station_loc, station_dt_w], axis=1).reshape(-1)
    ev2 = event_index.reshape(-1, _CH)
    out = _tt_pallas(station_index, ev2, phase_type, table, sta_tab)
    return out[:, None]


# speedup vs baseline: 73.7782x; 1.2627x over previous
import functools

import jax
import jax.numpy as jnp
from jax import lax
from jax.experimental import pallas as pl
from jax.experimental.pallas import tpu as pltpu
from jax.experimental.pallas import tpu_sc as plsc

_VP = 6.0
_VS = 6.0 / 1.73

_BLK = 2000
_CH = 100
_NCH = _BLK // _CH
_LANES = 16
_NW = 32


def _tt_body(sta_i_hbm, ev_i_hbm, ph_hbm, table_hbm, sta_tab_hbm, out_hbm,
             evi0, evi1, evi2, stai0, stai1, stai2, ph0, ph1, ph2,
             rows0, rows1, out0, out1, sta_v,
             semi0, semi1, semi2, semg0, semg1, semo0, semo1,
             *, num_blocks):
    wid = lax.axis_index("s") * 2 + lax.axis_index("c")
    evi = [evi0, evi1, evi2]
    stai = [stai0, stai1, stai2]
    phv = [ph0, ph1, ph2]
    rows = [rows0, rows1]
    outv = [out0, out1]
    semi = [semi0, semi1, semi2]
    semg = [semg0, semg1]
    semo = [semo0, semo1]

    pltpu.sync_copy(sta_tab_hbm, sta_v)

    nb = (num_blocks + _NW - 1) // _NW

    def blk_of(k):
        return wid + _NW * k

    def idx_copies(k):
        s = k % 3
        blk = blk_of(k)
        base = pl.multiple_of(blk * _BLK, 8)
        return [
            pltpu.make_async_copy(ev_i_hbm.at[pl.ds(blk * _NCH, _NCH)],
                                  evi[s], semi[s]),
            pltpu.make_async_copy(sta_i_hbm.at[pl.ds(base, _BLK)],
                                  stai[s], semi[s]),
            pltpu.make_async_copy(ph_hbm.at[pl.ds(base, _BLK)],
                                  phv[s], semi[s]),
        ]

    def gth_copies(k):
        s3, s2 = k % 3, k % 2
        return [
            pltpu.make_async_copy(table_hbm.at[evi[s3].at[j]],
                                  rows[s2].at[pl.ds(j * _CH, _CH)], semg[s2])
            for j in range(_NCH)
        ]

    def out_copy(k):
        s2 = k % 2
        base = pl.multiple_of(blk_of(k) * _BLK, 8)
        return pltpu.make_async_copy(outv[s2],
                                     out_hbm.at[pl.ds(base, _BLK)], semo[s2])

    def issue_idx(k):
        if not 0 <= k < nb:
            return

        @pl.when(blk_of(k) < num_blocks)
        def _():
            for c in idx_copies(k):
                c.start()

    def issue_gth(k):
        if not 0 <= k < nb:
            return

        @pl.when(blk_of(k) < num_blocks)
        def _():
            for c in idx_copies(k):
                c.wait()
            for c in gth_copies(k):
                c.start()

    def wait_gth(k):
        if not 0 <= k < nb:
            return

        @pl.when(blk_of(k) < num_blocks)
        def _():
            for c in gth_copies(k):
                c.wait()

    def issue_out(k):
        if not 0 <= k < nb:
            return

        @pl.when(blk_of(k) < num_blocks)
        def _():
            out_copy(k).start()

    def wait_out(k):
        if not 0 <= k < nb:
            return

        @pl.when(blk_of(k) < num_blocks)
        def _():
            out_copy(k).wait()

    def compute(k):
        if not 0 <= k < nb:
            return
        s3, s2 = k % 3, k % 2
        stai_v, ph_v, rows_v, out_v = stai[s3], phv[s3], rows[s2], outv[s2]

        @pl.when(blk_of(k) < num_blocks)
        def _():
            def group_step(g, carry):
                lane = lax.iota(jnp.int32, _LANES)
                rows16 = g * _LANES + lane
                c0 = jnp.zeros((_LANES,), jnp.int32)
                sta16 = stai_v[pl.ds(g * _LANES, _LANES)]
                ph16 = ph_v[pl.ds(g * _LANES, _LANES)]
                sbase = sta16 * 4
                ex = plsc.load_gather(rows_v, [rows16, c0])
                ey = plsc.load_gather(rows_v, [rows16, c0 + 1])
                ez = plsc.load_gather(rows_v, [rows16, c0 + 2])
                et = plsc.load_gather(rows_v, [rows16, c0 + 3])
                sx = plsc.load_gather(sta_v, [sbase])
                sy = plsc.load_gather(sta_v, [sbase + 1])
                sz = plsc.load_gather(sta_v, [sbase + 2])
                sdt = plsc.load_gather(sta_v, [sbase + 3])
                dx = ex - sx
                dy = ey - sy
                dz = ez - sz
                d2 = dx * dx + dy * dy + dz * dz
                seed = lax.bitcast_convert_type(
                    0x5F3759DF - lax.shift_right_logical(
                        lax.bitcast_convert_type(d2, jnp.int32), 1),
                    jnp.float32)
                y = seed * (1.5 - 0.5 * d2 * seed * seed)
                y = y * (1.5 - 0.5 * d2 * y * y)
                y = y * (1.5 - 0.5 * d2 * y * y)
                dist = jnp.where(d2 > 0.0, d2 * y, 0.0)
                inv_v = (1.0 / _VP) + ph16.astype(jnp.float32) * (1.0 / _VS - 1.0 / _VP)
                out_v[pl.ds(g * _LANES, _LANES)] = et + dist * inv_v + sdt
                return carry

            lax.fori_loop(0, _BLK // _LANES, group_step, 0, unroll=5)

    issue_idx(0)
    issue_gth(0)
    issue_idx(1)
    for k in range(nb):
        issue_gth(k + 1)
        wait_gth(k)
        issue_idx(k + 2)
        wait_out(k - 2)
        compute(k)
        issue_out(k)
    wait_out(nb - 2)
    wait_out(nb - 1)


@jax.jit
def _tt_pallas(station_index, ev2, phase_type, table, sta_tab):
    n = station_index.shape[0]
    assert n % _BLK == 0
    num_blocks = n // _BLK
    mesh = plsc.VectorSubcoreMesh(core_axis_name="c", subcore_axis_name="s")
    body = functools.partial(_tt_body, num_blocks=num_blocks)
    return pl.kernel(
        body,
        mesh=mesh,
        compiler_params=pltpu.CompilerParams(needs_layout_passes=False,
                                             use_tc_tiling_on_sc=False),
        out_type=jax.ShapeDtypeStruct((n,), jnp.float32),
        scratch_types=[
            pltpu.VMEM((_NCH, _CH), jnp.int32),
            pltpu.VMEM((_NCH, _CH), jnp.int32),
            pltpu.VMEM((_NCH, _CH), jnp.int32),
            pltpu.VMEM((_BLK,), jnp.int32),
            pltpu.VMEM((_BLK,), jnp.int32),
            pltpu.VMEM((_BLK,), jnp.int32),
            pltpu.VMEM((_BLK,), jnp.int32),
            pltpu.VMEM((_BLK,), jnp.int32),
            pltpu.VMEM((_BLK,), jnp.int32),
            pltpu.VMEM((_BLK, 16), jnp.float32),
            pltpu.VMEM((_BLK, 16), jnp.float32),
            pltpu.VMEM((_BLK,), jnp.float32),
            pltpu.VMEM((_BLK,), jnp.float32),
            pltpu.VMEM((400,), jnp.float32),
            pltpu.SemaphoreType.DMA,
            pltpu.SemaphoreType.DMA,
            pltpu.SemaphoreType.DMA,
            pltpu.SemaphoreType.DMA,
            pltpu.SemaphoreType.DMA,
            pltpu.SemaphoreType.DMA,
            pltpu.SemaphoreType.DMA,
        ],
    )(station_index, ev2, phase_type, table, sta_tab)


def kernel(station_index, event_index, phase_type, event_loc_w, event_time_w,
           station_dt_w, station_loc):
    table = jnp.pad(jnp.concatenate([event_loc_w, event_time_w], axis=1),
                    ((0, 0), (0, 12)))
    sta_tab = jnp.concatenate([station_loc, station_dt_w], axis=1).reshape(-1)
    ev2 = event_index.reshape(-1, _CH)
    out = _tt_pallas(station_index, ev2, phase_type, table, sta_tab)
    return out[:, None]
